# Initial kernel scaffold; baseline (speedup 1.0000x reference)
import jax
import jax.numpy as jnp
from jax.experimental import pallas as pl


def kernel(x, edge_index, W1l, W1r, b1, W2l, W2r, b2):
    raise NotImplementedError("write your pallas kernel here")


# speedup vs baseline: 5.6885x; 5.6885x over previous
import functools

import jax
import jax.numpy as jnp
from jax import lax
from jax.experimental import pallas as pl
from jax.experimental.pallas import tpu as pltpu
from jax.experimental.pallas import tpu_sc as plsc

N = 10000
E = 320000
D = 128

NC = 2
NS = 16
NW = NC * NS
CHUNK = 128
CW = -(-E // (NW * CHUNK))
EPW = CW * CHUNK
E_PAD = NW * EPW
RPS = 640
N_PAD = NS * RPS
RB = RPS // CHUNK


def _mesh():
    return plsc.VectorSubcoreMesh(
        core_axis_name="c", subcore_axis_name="s",
        num_cores=NC, num_subcores=NS,
    )


@functools.cache
def _sc_agg():
    def body(x_hbm, src_hbm, dst_hbm, zr_hbm, iota_hbm,
             agg_hbm, src_v, dst_v, rows_v, agg_sh):
        c = lax.axis_index("c")
        s = lax.axis_index("s")
        w = c * NS + s
        r0 = s * RPS

        pltpu.sync_copy(zr_hbm, rows_v)
        for b in range(RB):
            pltpu.sync_copy(iota_hbm.at[pl.ds(r0 + b * CHUNK, CHUNK)], dst_v)
            pltpu.sync_copy(rows_v, agg_sh.at[dst_v])
        plsc.subcore_barrier()

        e0 = w * EPW

        def step(j, carry):
            pltpu.sync_copy(src_hbm.at[pl.ds(e0 + j * CHUNK, CHUNK)], src_v)
            pltpu.sync_copy(dst_hbm.at[pl.ds(e0 + j * CHUNK, CHUNK)], dst_v)
            pltpu.sync_copy(x_hbm.at[src_v], rows_v)
            pltpu.sync_copy(rows_v, agg_sh.at[dst_v], add=True)
            return carry

        lax.fori_loop(0, CW, step, 0)
        plsc.subcore_barrier()

        for b in range(RB):
            pltpu.sync_copy(iota_hbm.at[pl.ds(r0 + b * CHUNK, CHUNK)], dst_v)
            pltpu.sync_copy(agg_sh.at[dst_v], rows_v)
            pltpu.sync_copy(rows_v, agg_hbm.at[c, pl.ds(r0 + b * CHUNK, CHUNK)])

    return pl.kernel(
        body,
        out_type=[jax.ShapeDtypeStruct((NC, N_PAD, D), jnp.float32)],
        mesh=_mesh(),
        scratch_types=[
            pltpu.VMEM((CHUNK,), jnp.int32),
            pltpu.VMEM((CHUNK,), jnp.int32),
            pltpu.VMEM((CHUNK, D), jnp.float32),
            pltpu.VMEM_SHARED((N_PAD, D), jnp.float32),
        ],
    )


@functools.cache
def _sc_cnt():
    def body(dst_hbm, zr_hbm, ones_hbm, iota_hbm,
             cnt_hbm, dst_v, rows_v, cnt_sh):
        c = lax.axis_index("c")
        s = lax.axis_index("s")
        w = c * NS + s
        r0 = s * RPS

        pltpu.sync_copy(zr_hbm, rows_v)
        for b in range(RB):
            pltpu.sync_copy(iota_hbm.at[pl.ds(r0 + b * CHUNK, CHUNK)], dst_v)
            pltpu.sync_copy(rows_v, cnt_sh.at[dst_v])
        pltpu.sync_copy(ones_hbm, rows_v)
        plsc.subcore_barrier()

        e0 = w * EPW

        def step(j, carry):
            pltpu.sync_copy(dst_hbm.at[pl.ds(e0 + j * CHUNK, CHUNK)], dst_v)
            pltpu.sync_copy(rows_v, cnt_sh.at[dst_v], add=True)
            return carry

        lax.fori_loop(0, CW, step, 0)
        plsc.subcore_barrier()

        for b in range(RB):
            pltpu.sync_copy(iota_hbm.at[pl.ds(r0 + b * CHUNK, CHUNK)], dst_v)
            pltpu.sync_copy(cnt_sh.at[dst_v], rows_v)
            pltpu.sync_copy(rows_v, cnt_hbm.at[c, pl.ds(r0 + b * CHUNK, CHUNK)])

    return pl.kernel(
        body,
        out_type=[jax.ShapeDtypeStruct((NC, N_PAD, D), jnp.float32)],
        mesh=_mesh(),
        scratch_types=[
            pltpu.VMEM((CHUNK,), jnp.int32),
            pltpu.VMEM((CHUNK, D), jnp.float32),
            pltpu.VMEM_SHARED((N_PAD, D), jnp.float32),
        ],
    )


def _tc_dense_body(relu, agg_ref, cnt_ref, x_ref, wl_ref, wr_ref, b_ref, o_ref):
    agg = agg_ref[0] + agg_ref[1]
    cnt = cnt_ref[0, :, 0:1] + cnt_ref[1, :, 0:1]
    mean = agg / jnp.maximum(cnt, 1.0)
    acc = jnp.dot(mean, wl_ref[...], preferred_element_type=jnp.float32)
    acc += jnp.dot(x_ref[...], wr_ref[...], preferred_element_type=jnp.float32)
    acc += b_ref[...]
    if relu:
        acc = jnp.maximum(acc, 0.0)
    o_ref[...] = acc


_TC_BLK = 1000


def _tc_dense(agg, cnt, x, wl, wr, b, relu):
    grid = N // _TC_BLK
    return pl.pallas_call(
        functools.partial(_tc_dense_body, relu),
        grid=(grid,),
        in_specs=[
            pl.BlockSpec((NC, _TC_BLK, D), lambda i: (0, i, 0)),
            pl.BlockSpec((NC, _TC_BLK, D), lambda i: (0, i, 0)),
            pl.BlockSpec((_TC_BLK, D), lambda i: (i, 0)),
            pl.BlockSpec((D, D), lambda i: (0, 0)),
            pl.BlockSpec((D, D), lambda i: (0, 0)),
            pl.BlockSpec((1, D), lambda i: (0, 0)),
        ],
        out_specs=pl.BlockSpec((_TC_BLK, D), lambda i: (i, 0)),
        out_shape=jax.ShapeDtypeStruct((N, D), jnp.float32),
    )(agg, cnt, x, wl, wr, b)


@jax.jit
def kernel(x, edge_index, W1l, W1r, b1, W2l, W2r, b2):
    src = edge_index[0]
    dst = edge_index[1]
    pad = E_PAD - E
    pad_ids = jnp.arange(pad, dtype=jnp.int32)
    src_p = jnp.concatenate([src, pad_ids % N])
    dst_p = jnp.concatenate([dst, N + (pad_ids % (N_PAD - N))])

    zr = jnp.zeros((CHUNK, D), jnp.float32)
    onesr = jnp.ones((CHUNK, D), jnp.float32)
    iota = jnp.arange(N_PAD, dtype=jnp.int32)

    (cnt,) = _sc_cnt()(dst_p, zr, onesr, iota)
    (agg1,) = _sc_agg()(x, src_p, dst_p, zr, iota)
    h = _tc_dense(agg1, cnt, x, W1l, W1r, b1.reshape(1, D), relu=True)
    (agg2,) = _sc_agg()(h, src_p, dst_p, zr, iota)
    out = _tc_dense(agg2, cnt, h, W2l, W2r, b2.reshape(1, D), relu=False)
    return out


# speedup vs baseline: 7.9570x; 1.3988x over previous
import functools

import jax
import jax.numpy as jnp
from jax import lax
from jax.experimental import pallas as pl
from jax.experimental.pallas import tpu as pltpu
from jax.experimental.pallas import tpu_sc as plsc

N = 10000
E = 320000
D = 128

NC = 2
NS = 16
NW = NC * NS
CHUNK = 128
CW = -(-E // (NW * CHUNK))
CW += CW % 2
EPW = CW * CHUNK
E_PAD = NW * EPW
RPS = 640
N_PAD = NS * RPS
RB = RPS // CHUNK


def _mesh():
    return plsc.VectorSubcoreMesh(
        core_axis_name="c", subcore_axis_name="s",
        num_cores=NC, num_subcores=NS,
    )


@functools.cache
def _sc_agg():
    def body(x_hbm, src_hbm, dst_hbm, zr_hbm, iota_hbm, agg_hbm,
             src_a, dst_a, src_b, dst_b, rows_a, rows_b, sem_a, sem_b,
             agg_sh):
        c = lax.axis_index("c")
        s = lax.axis_index("s")
        w = c * NS + s
        r0 = s * RPS

        pltpu.sync_copy(zr_hbm, rows_a)
        for b in range(RB):
            pltpu.sync_copy(iota_hbm.at[pl.ds(r0 + b * CHUNK, CHUNK)], dst_a)
            pltpu.sync_copy(rows_a, agg_sh.at[dst_a])
        plsc.subcore_barrier()

        e0 = w * EPW

        def _stage(j, src_v, dst_v):
            pltpu.sync_copy(src_hbm.at[pl.ds(e0 + j * CHUNK, CHUNK)], src_v)
            pltpu.sync_copy(dst_hbm.at[pl.ds(e0 + j * CHUNK, CHUNK)], dst_v)

        _stage(0, src_a, dst_a)
        pltpu.async_copy(x_hbm.at[src_a], rows_a, sem_a)

        def pair(k, carry):
            _stage(2 * k + 1, src_b, dst_b)
            pltpu.async_copy(x_hbm.at[src_b], rows_b, sem_b)
            pltpu.make_async_copy(x_hbm.at[src_a], rows_a, sem_a).wait()
            pltpu.sync_copy(rows_a, agg_sh.at[dst_a], add=True)

            @pl.when(k < CW // 2 - 1)
            def _():
                _stage(2 * k + 2, src_a, dst_a)
                pltpu.async_copy(x_hbm.at[src_a], rows_a, sem_a)

            pltpu.make_async_copy(x_hbm.at[src_b], rows_b, sem_b).wait()
            pltpu.sync_copy(rows_b, agg_sh.at[dst_b], add=True)
            return carry

        lax.fori_loop(0, CW // 2, pair, 0)
        plsc.subcore_barrier()

        for b in range(RB):
            pltpu.sync_copy(iota_hbm.at[pl.ds(r0 + b * CHUNK, CHUNK)], dst_a)
            pltpu.sync_copy(agg_sh.at[dst_a], rows_a)
            pltpu.sync_copy(rows_a, agg_hbm.at[c, pl.ds(r0 + b * CHUNK, CHUNK)])

    return pl.kernel(
        body,
        out_type=[jax.ShapeDtypeStruct((NC, N_PAD, D), jnp.float32)],
        mesh=_mesh(),
        scratch_types=[
            pltpu.VMEM((CHUNK,), jnp.int32),
            pltpu.VMEM((CHUNK,), jnp.int32),
            pltpu.VMEM((CHUNK,), jnp.int32),
            pltpu.VMEM((CHUNK,), jnp.int32),
            pltpu.VMEM((CHUNK, D), jnp.float32),
            pltpu.VMEM((CHUNK, D), jnp.float32),
            pltpu.SemaphoreType.DMA,
            pltpu.SemaphoreType.DMA,
            pltpu.VMEM_SHARED((N_PAD, D), jnp.float32),
        ],
    )


@functools.cache
def _sc_cnt():
    def body(dst_hbm, zr_hbm, ones_hbm, iota_hbm,
             cnt_hbm, dst_v, rows_v, cnt_sh):
        c = lax.axis_index("c")
        s = lax.axis_index("s")
        w = c * NS + s
        r0 = s * RPS

        pltpu.sync_copy(zr_hbm, rows_v)
        for b in range(RB):
            pltpu.sync_copy(iota_hbm.at[pl.ds(r0 + b * CHUNK, CHUNK)], dst_v)
            pltpu.sync_copy(rows_v, cnt_sh.at[dst_v])
        pltpu.sync_copy(ones_hbm, rows_v)
        plsc.subcore_barrier()

        e0 = w * EPW

        def step(j, carry):
            pltpu.sync_copy(dst_hbm.at[pl.ds(e0 + j * CHUNK, CHUNK)], dst_v)
            pltpu.sync_copy(rows_v, cnt_sh.at[dst_v], add=True)
            return carry

        lax.fori_loop(0, CW, step, 0)
        plsc.subcore_barrier()

        for b in range(RB):
            pltpu.sync_copy(iota_hbm.at[pl.ds(r0 + b * CHUNK, CHUNK)], dst_v)
            pltpu.sync_copy(cnt_sh.at[dst_v], rows_v)
            pltpu.sync_copy(rows_v, cnt_hbm.at[c, pl.ds(r0 + b * CHUNK, CHUNK)])

    return pl.kernel(
        body,
        out_type=[jax.ShapeDtypeStruct((NC, N_PAD, D), jnp.float32)],
        mesh=_mesh(),
        scratch_types=[
            pltpu.VMEM((CHUNK,), jnp.int32),
            pltpu.VMEM((CHUNK, D), jnp.float32),
            pltpu.VMEM_SHARED((N_PAD, D), jnp.float32),
        ],
    )


def _tc_dense_body(relu, agg_ref, cnt_ref, x_ref, wl_ref, wr_ref, b_ref, o_ref):
    agg = agg_ref[0] + agg_ref[1]
    cnt = cnt_ref[0, :, 0:1] + cnt_ref[1, :, 0:1]
    mean = agg / jnp.maximum(cnt, 1.0)
    acc = jnp.dot(mean, wl_ref[...], preferred_element_type=jnp.float32)
    acc += jnp.dot(x_ref[...], wr_ref[...], preferred_element_type=jnp.float32)
    acc += b_ref[...]
    if relu:
        acc = jnp.maximum(acc, 0.0)
    o_ref[...] = acc


_TC_BLK = 1000


def _tc_dense(agg, cnt, x, wl, wr, b, relu):
    grid = N // _TC_BLK
    return pl.pallas_call(
        functools.partial(_tc_dense_body, relu),
        grid=(grid,),
        in_specs=[
            pl.BlockSpec((NC, _TC_BLK, D), lambda i: (0, i, 0)),
            pl.BlockSpec((NC, _TC_BLK, D), lambda i: (0, i, 0)),
            pl.BlockSpec((_TC_BLK, D), lambda i: (i, 0)),
            pl.BlockSpec((D, D), lambda i: (0, 0)),
            pl.BlockSpec((D, D), lambda i: (0, 0)),
            pl.BlockSpec((1, D), lambda i: (0, 0)),
        ],
        out_specs=pl.BlockSpec((_TC_BLK, D), lambda i: (i, 0)),
        out_shape=jax.ShapeDtypeStruct((N, D), jnp.float32),
    )(agg, cnt, x, wl, wr, b)


@jax.jit
def kernel(x, edge_index, W1l, W1r, b1, W2l, W2r, b2):
    src = edge_index[0]
    dst = edge_index[1]
    pad = E_PAD - E
    pad_ids = jnp.arange(pad, dtype=jnp.int32)
    src_p = jnp.concatenate([src, pad_ids % N])
    dst_p = jnp.concatenate([dst, N + (pad_ids % (N_PAD - N))])

    zr = jnp.zeros((CHUNK, D), jnp.float32)
    onesr = jnp.ones((CHUNK, D), jnp.float32)
    iota = jnp.arange(N_PAD, dtype=jnp.int32)

    (cnt,) = _sc_cnt()(dst_p, zr, onesr, iota)
    (agg1,) = _sc_agg()(x, src_p, dst_p, zr, iota)
    h = _tc_dense(agg1, cnt, x, W1l, W1r, b1.reshape(1, D), relu=True)
    (agg2,) = _sc_agg()(h, src_p, dst_p, zr, iota)
    out = _tc_dense(agg2, cnt, h, W2l, W2r, b2.reshape(1, D), relu=False)
    return out


# speedup vs baseline: 9.2449x; 1.1619x over previous
import functools

import jax
import jax.numpy as jnp
from jax import lax
from jax.experimental import pallas as pl
from jax.experimental.pallas import tpu as pltpu
from jax.experimental.pallas import tpu_sc as plsc

N = 10000
E = 320000
D = 128

NC = 2
NS = 16
NW = NC * NS
CHUNK = 128
CW = -(-E // (NW * CHUNK))
CW += CW % 2
EPW = CW * CHUNK
E_PAD = NW * EPW
RPS = 640
N_PAD = NS * RPS
RB = RPS // CHUNK
G = 8
NG = CW // G


def _mesh():
    return plsc.VectorSubcoreMesh(
        core_axis_name="c", subcore_axis_name="s",
        num_cores=NC, num_subcores=NS,
    )


@functools.cache
def _sc_agg():
    def body(x_hbm, src_hbm, dst_hbm, zr_hbm, iota_hbm, agg_hbm,
             src_a, dst_a, idx_v, rows_a, rows_b, sem_a, sem_b, agg_sh):
        c = lax.axis_index("c")
        s = lax.axis_index("s")
        w = c * NS + s
        r0 = s * RPS

        pltpu.sync_copy(zr_hbm, rows_a)
        for b in range(RB):
            pltpu.sync_copy(iota_hbm.at[pl.ds(r0 + b * CHUNK, CHUNK)], idx_v)
            pltpu.sync_copy(rows_a, agg_sh.at[idx_v])
        plsc.subcore_barrier()

        def group(g, carry):
            pltpu.sync_copy(src_hbm.at[w * NG + g], src_a)
            pltpu.sync_copy(dst_hbm.at[w * NG + g], dst_a)
            pltpu.async_copy(x_hbm.at[src_a.at[0]], rows_a, sem_a)
            for m in range(G):
                buf, sem = (rows_a, sem_a) if m % 2 == 0 else (rows_b, sem_b)
                if m < G - 1:
                    nbuf, nsem = ((rows_b, sem_b) if m % 2 == 0
                                  else (rows_a, sem_a))
                    pltpu.async_copy(x_hbm.at[src_a.at[m + 1]], nbuf, nsem)
                pltpu.make_async_copy(x_hbm.at[src_a.at[m]], buf, sem).wait()
                pltpu.sync_copy(buf, agg_sh.at[dst_a.at[m]], add=True)
            return carry

        lax.fori_loop(0, NG, group, 0)
        plsc.subcore_barrier()

        for b in range(RB):
            pltpu.sync_copy(iota_hbm.at[pl.ds(r0 + b * CHUNK, CHUNK)], idx_v)
            pltpu.sync_copy(agg_sh.at[idx_v], rows_a)
            pltpu.sync_copy(rows_a, agg_hbm.at[c, pl.ds(r0 + b * CHUNK, CHUNK)])

    return pl.kernel(
        body,
        out_type=[jax.ShapeDtypeStruct((NC, N_PAD, D), jnp.float32)],
        mesh=_mesh(),
        scratch_types=[
            pltpu.VMEM((G, CHUNK), jnp.int32),
            pltpu.VMEM((G, CHUNK), jnp.int32),
            pltpu.VMEM((CHUNK,), jnp.int32),
            pltpu.VMEM((CHUNK, D), jnp.float32),
            pltpu.VMEM((CHUNK, D), jnp.float32),
            pltpu.SemaphoreType.DMA,
            pltpu.SemaphoreType.DMA,
            pltpu.VMEM_SHARED((N_PAD, D), jnp.float32),
        ],
    )


@functools.cache
def _sc_cnt():
    def body(dst_hbm, zr_hbm, ones_hbm, iota_hbm,
             cnt_hbm, dst8, idx_v, rows_v, sem, cnt_sh):
        c = lax.axis_index("c")
        s = lax.axis_index("s")
        w = c * NS + s
        r0 = s * RPS

        pltpu.sync_copy(zr_hbm, rows_v)
        for b in range(RB):
            pltpu.sync_copy(iota_hbm.at[pl.ds(r0 + b * CHUNK, CHUNK)], idx_v)
            pltpu.sync_copy(rows_v, cnt_sh.at[idx_v])
        pltpu.sync_copy(ones_hbm, rows_v)
        plsc.subcore_barrier()

        def group(g, carry):
            pltpu.sync_copy(dst_hbm.at[w * NG + g], dst8)
            for m in range(G):
                pltpu.async_copy(rows_v, cnt_sh.at[dst8.at[m]], sem, add=True)
            for m in range(G):
                pltpu.make_async_copy(
                    rows_v, cnt_sh.at[dst8.at[m]], sem).wait()
            return carry

        lax.fori_loop(0, NG, group, 0)
        plsc.subcore_barrier()

        for b in range(RB):
            pltpu.sync_copy(iota_hbm.at[pl.ds(r0 + b * CHUNK, CHUNK)], idx_v)
            pltpu.sync_copy(cnt_sh.at[idx_v], rows_v)
            pltpu.sync_copy(rows_v, cnt_hbm.at[c, pl.ds(r0 + b * CHUNK, CHUNK)])

    return pl.kernel(
        body,
        out_type=[jax.ShapeDtypeStruct((NC, N_PAD, D), jnp.float32)],
        mesh=_mesh(),
        scratch_types=[
            pltpu.VMEM((G, CHUNK), jnp.int32),
            pltpu.VMEM((CHUNK,), jnp.int32),
            pltpu.VMEM((CHUNK, D), jnp.float32),
            pltpu.SemaphoreType.DMA,
            pltpu.VMEM_SHARED((N_PAD, D), jnp.float32),
        ],
    )


def _tc_dense_body(relu, agg_ref, cnt_ref, x_ref, wl_ref, wr_ref, b_ref, o_ref):
    agg = agg_ref[0] + agg_ref[1]
    cnt = cnt_ref[0, :, 0:1] + cnt_ref[1, :, 0:1]
    mean = agg / jnp.maximum(cnt, 1.0)
    acc = jnp.dot(mean, wl_ref[...], preferred_element_type=jnp.float32)
    acc += jnp.dot(x_ref[...], wr_ref[...], preferred_element_type=jnp.float32)
    acc += b_ref[...]
    if relu:
        acc = jnp.maximum(acc, 0.0)
    o_ref[...] = acc


_TC_BLK = 1000


def _tc_dense(agg, cnt, x, wl, wr, b, relu):
    grid = N // _TC_BLK
    return pl.pallas_call(
        functools.partial(_tc_dense_body, relu),
        grid=(grid,),
        in_specs=[
            pl.BlockSpec((NC, _TC_BLK, D), lambda i: (0, i, 0)),
            pl.BlockSpec((NC, _TC_BLK, D), lambda i: (0, i, 0)),
            pl.BlockSpec((_TC_BLK, D), lambda i: (i, 0)),
            pl.BlockSpec((D, D), lambda i: (0, 0)),
            pl.BlockSpec((D, D), lambda i: (0, 0)),
            pl.BlockSpec((1, D), lambda i: (0, 0)),
        ],
        out_specs=pl.BlockSpec((_TC_BLK, D), lambda i: (i, 0)),
        out_shape=jax.ShapeDtypeStruct((N, D), jnp.float32),
    )(agg, cnt, x, wl, wr, b)


@jax.jit
def kernel(x, edge_index, W1l, W1r, b1, W2l, W2r, b2):
    src = edge_index[0]
    dst = edge_index[1]
    pad = E_PAD - E
    pad_ids = jnp.arange(pad, dtype=jnp.int32)
    src_p = jnp.concatenate([src, pad_ids % N]).reshape(NW * NG, G, CHUNK)
    dst_p = jnp.concatenate([dst, N + (pad_ids % (N_PAD - N))]).reshape(
        NW * NG, G, CHUNK)

    zr = jnp.zeros((CHUNK, D), jnp.float32)
    onesr = jnp.ones((CHUNK, D), jnp.float32)
    iota = jnp.arange(N_PAD, dtype=jnp.int32)

    (cnt,) = _sc_cnt()(dst_p, zr, onesr, iota)
    (agg1,) = _sc_agg()(x, src_p, dst_p, zr, iota)
    h = _tc_dense(agg1, cnt, x, W1l, W1r, b1.reshape(1, D), relu=True)
    (agg2,) = _sc_agg()(h, src_p, dst_p, zr, iota)
    out = _tc_dense(agg2, cnt, h, W2l, W2r, b2.reshape(1, D), relu=False)
    return out
